# Initial kernel scaffold; baseline (speedup 1.0000x reference)
import jax
import jax.numpy as jnp
from jax.experimental import pallas as pl


def kernel(features, depth, lidar2img):
    raise NotImplementedError("write your pallas kernel here")


# speedup vs baseline: 13.9047x; 13.9047x over previous
import functools

import jax
import jax.numpy as jnp
from jax import lax
from jax.experimental import pallas as pl
from jax.experimental.pallas import tpu as pltpu
from jax.experimental.pallas import tpu_sc as plsc

F32 = jnp.float32
I32 = jnp.int32

BN = 12
D = 41
FH, FW = 28, 60
HW = FH * FW
C = 64
P = BN * D * HW
Q = BN * HW

NQ = 4
CQ = C // NQ
NV = 80000
MC = 2048
SUB = 128
NSUB = MC // SUB
SHARD_MC = 26
SHARD = SHARD_MC * MC
P_PAD = 16 * SHARD
SLICE = NV // 16


def _tc_body(gx_ref, gy_ref, gz_ref, depth_ref, wt_ref, rk_ref, px_ref):
    i = pl.program_id(0)
    d = depth_ref[0]
    mx = jnp.max(d, axis=0, keepdims=True)
    e = jnp.exp(d - mx)
    wt = e / jnp.sum(e, axis=0, keepdims=True)

    vx = ((gx_ref[0] + 50.0) / 0.5).astype(I32)
    vy = ((gy_ref[0] + 50.0) / 0.5).astype(I32)
    vz = ((gz_ref[0] + 10.0) / 20.0).astype(I32)
    kept = ((vx >= 0) & (vx < 200) & (vy >= 0) & (vy < 200)
            & (vz >= 0) & (vz < 1))
    rank = vx * 400 + vy * 2 + i // 6
    rk_ref[0] = jnp.where(kept, rank, 0)
    wt_ref[0] = jnp.where(kept, wt, 0.0)
    px_ref[0] = lax.broadcasted_iota(I32, (D, HW), 1) + i * HW


def _tc_phase(gx, gy, gz, depth):
    bs = pl.BlockSpec((1, D, HW), lambda i: (i, 0, 0))
    return pl.pallas_call(
        _tc_body,
        grid=(BN,),
        in_specs=[bs, bs, bs, bs],
        out_specs=[bs, bs, bs],
        out_shape=[
            jax.ShapeDtypeStruct((BN, D, HW), F32),
            jax.ShapeDtypeStruct((BN, D, HW), I32),
            jax.ShapeDtypeStruct((BN, D, HW), I32),
        ],
    )(gx, gy, gz, depth)


def _geometry(lidar2img):
    ds = jnp.arange(4.0, 45.0, 1.0, dtype=F32).reshape(-1, 1, 1)
    ds = jnp.broadcast_to(ds, (D, FH, FW))
    xs = jnp.broadcast_to(
        jnp.linspace(0.0, 479.0, FW, dtype=F32).reshape(1, 1, FW), (D, FH, FW))
    ys = jnp.broadcast_to(
        jnp.linspace(0.0, 223.0, FH, dtype=F32).reshape(1, FH, 1), (D, FH, FW))
    frustum = jnp.stack((xs, ys, ds), -1)
    eps = 1e-05
    pts = jnp.broadcast_to(frustum[None, None], (2, 6, D, FH, FW, 3))
    pts = jnp.concatenate([pts, jnp.ones_like(pts[..., :1])], -1)
    xy = pts[..., :2] * jnp.maximum(pts[..., 2:3], jnp.ones_like(pts[..., 2:3]) * eps)
    pts = jnp.concatenate([xy, pts[..., 2:]], -1)
    img2lidars = jnp.linalg.inv(lidar2img)
    geom = jnp.einsum('bnij,bndhwj->bndhwi', img2lidars, pts)[..., :3]
    g = geom.reshape(BN, D, HW, 3)
    return g[..., 0], g[..., 1], g[..., 2]


def _sc_body(rk_hbm, wt_hbm, px_hbm, ftq_hbm, z_hbm, out_hbm,
             rk_mc, wt_mc, px_mc, rk128, ix128, rows, acc, sem):
    core = lax.axis_index("c")
    s = lax.axis_index("s")
    base_pt = s * SHARD

    for qq in range(NQ // 2):
        q = core * 2 + qq
        qoff = q * Q
        pltpu.sync_copy(z_hbm, acc.at[pl.ds(s * SLICE, SLICE)])
        plsc.subcore_barrier()

        def chunk_body(ch, _):
            cbase = base_pt + ch * MC
            pltpu.sync_copy(rk_hbm.at[pl.ds(cbase, MC)], rk_mc)
            pltpu.sync_copy(wt_hbm.at[pl.ds(cbase, MC)], wt_mc)
            pltpu.sync_copy(px_hbm.at[pl.ds(cbase, MC)], px_mc)
            for sub in range(NSUB):
                sb = sub * SUB
                for r in range(SUB // 16):
                    sl16 = pl.ds(sb + r * 16, 16)
                    rk128[pl.ds(r * 16, 16)] = rk_mc[sl16]
                    ix128[pl.ds(r * 16, 16)] = px_mc[sl16] + qoff
                pltpu.async_copy(ftq_hbm.at[ix128], rows, sem).wait()

                def srow(r, _):
                    wr = plsc.load_gather(
                        wt_mc, [jnp.full((16,), sb + r, I32)])
                    rows[r, pl.ds(0, 16)] = rows[r, pl.ds(0, 16)] * wr
                    return 0

                lax.fori_loop(0, SUB, srow, 0)
                pltpu.sync_copy(rows, acc.at[rk128], add=True)
            return 0

        lax.fori_loop(0, SHARD_MC, chunk_body, 0)
        plsc.subcore_barrier()
        pltpu.sync_copy(acc.at[pl.ds(s * SLICE, SLICE)],
                        out_hbm.at[q, pl.ds(s * SLICE, SLICE)])


@functools.cache
def _sc_scatter():
    return pl.kernel(
        _sc_body,
        out_type=jax.ShapeDtypeStruct((NQ, NV, CQ), F32),
        mesh=plsc.VectorSubcoreMesh(core_axis_name="c", subcore_axis_name="s"),
        scratch_types=[
            pltpu.VMEM((MC,), I32),
            pltpu.VMEM((MC,), F32),
            pltpu.VMEM((MC,), I32),
            pltpu.VMEM((SUB,), I32),
            pltpu.VMEM((SUB,), I32),
            pltpu.VMEM((SUB, CQ), F32),
            pltpu.VMEM_SHARED((NV, CQ), F32),
            pltpu.SemaphoreType.DMA,
        ],
        compiler_params=pltpu.CompilerParams(
            needs_layout_passes=False, use_tc_tiling_on_sc=False),
    )


def kernel(features, depth, lidar2img):
    gx, gy, gz = _geometry(lidar2img)
    depth_r = depth.reshape(BN, D, HW)
    wt, rk, px = _tc_phase(gx, gy, gz, depth_r)

    pad = P_PAD - P
    rk_f = jnp.concatenate([rk.reshape(P), jnp.zeros((pad,), I32)])
    wt_f = jnp.concatenate([wt.reshape(P), jnp.zeros((pad,), F32)])
    px_f = jnp.concatenate([px.reshape(P), jnp.zeros((pad,), I32)])
    ftq = (features.reshape(BN, C, HW).transpose(0, 2, 1).reshape(Q, NQ, CQ)
           .transpose(1, 0, 2).reshape(NQ * Q, CQ))
    zeros = jnp.zeros((SLICE, CQ), F32)

    out = _sc_scatter()(rk_f, wt_f, px_f, ftq, zeros)
    o = out.reshape(NQ, 200, 200, 2, CQ)
    return o.transpose(3, 0, 4, 1, 2).reshape(2, C, 200, 200)


# speedup vs baseline: 14.8311x; 1.0666x over previous
import functools

import jax
import jax.numpy as jnp
from jax import lax
from jax.experimental import pallas as pl
from jax.experimental.pallas import tpu as pltpu
from jax.experimental.pallas import tpu_sc as plsc

F32 = jnp.float32
I32 = jnp.int32

BN = 12
D = 41
FH, FW = 28, 60
HW = FH * FW
C = 64
P = BN * D * HW
Q = BN * HW

NQ = 4
CQ = C // NQ
NV = 80000
MC = 2048
SUB = 128
NSUB = MC // SUB
SHARD_MC = 26
SHARD = SHARD_MC * MC
P_PAD = 16 * SHARD
SLICE = NV // 16


def _tc_body(gx_ref, gy_ref, gz_ref, depth_ref, wt_ref, rk_ref, px_ref):
    i = pl.program_id(0)
    d = depth_ref[0]
    mx = jnp.max(d, axis=0, keepdims=True)
    e = jnp.exp(d - mx)
    wt = e / jnp.sum(e, axis=0, keepdims=True)

    vx = ((gx_ref[0] + 50.0) / 0.5).astype(I32)
    vy = ((gy_ref[0] + 50.0) / 0.5).astype(I32)
    vz = ((gz_ref[0] + 10.0) / 20.0).astype(I32)
    kept = ((vx >= 0) & (vx < 200) & (vy >= 0) & (vy < 200)
            & (vz >= 0) & (vz < 1))
    rank = vx * 400 + vy * 2 + i // 6
    rk_ref[0] = jnp.where(kept, rank, 0)
    wt_ref[0] = jnp.where(kept, wt, 0.0)
    px_ref[0] = lax.broadcasted_iota(I32, (D, HW), 1) + i * HW


def _tc_phase(gx, gy, gz, depth):
    bs = pl.BlockSpec((1, D, HW), lambda i: (i, 0, 0))
    return pl.pallas_call(
        _tc_body,
        grid=(BN,),
        in_specs=[bs, bs, bs, bs],
        out_specs=[bs, bs, bs],
        out_shape=[
            jax.ShapeDtypeStruct((BN, D, HW), F32),
            jax.ShapeDtypeStruct((BN, D, HW), I32),
            jax.ShapeDtypeStruct((BN, D, HW), I32),
        ],
    )(gx, gy, gz, depth)


def _geometry(lidar2img):
    ds = jnp.arange(4.0, 45.0, 1.0, dtype=F32).reshape(-1, 1, 1)
    ds = jnp.broadcast_to(ds, (D, FH, FW))
    xs = jnp.broadcast_to(
        jnp.linspace(0.0, 479.0, FW, dtype=F32).reshape(1, 1, FW), (D, FH, FW))
    ys = jnp.broadcast_to(
        jnp.linspace(0.0, 223.0, FH, dtype=F32).reshape(1, FH, 1), (D, FH, FW))
    frustum = jnp.stack((xs, ys, ds), -1)
    eps = 1e-05
    pts = jnp.broadcast_to(frustum[None, None], (2, 6, D, FH, FW, 3))
    pts = jnp.concatenate([pts, jnp.ones_like(pts[..., :1])], -1)
    xy = pts[..., :2] * jnp.maximum(pts[..., 2:3], jnp.ones_like(pts[..., 2:3]) * eps)
    pts = jnp.concatenate([xy, pts[..., 2:]], -1)
    img2lidars = jnp.linalg.inv(lidar2img)
    geom = jnp.einsum('bnij,bndhwj->bndhwi', img2lidars, pts)[..., :3]
    g = geom.reshape(BN, D, HW, 3)
    return g[..., 0], g[..., 1], g[..., 2]


def _sc_body(rk_hbm, wt_hbm, px_hbm, ftq_hbm, z_hbm, out_hbm,
             rk_mc, wt_mc, px_mc, rk128, ix128, rows, acc,
             gsem0, gsem1, ssem0, ssem1):
    core = lax.axis_index("c")
    s = lax.axis_index("s")
    base_pt = s * SHARD
    gsem = (gsem0, gsem1)
    ssem = (ssem0, ssem1)

    for qq in range(NQ // 2):
        q = core * 2 + qq
        qoff = q * Q
        pltpu.sync_copy(z_hbm, acc.at[pl.ds(s * SLICE, SLICE)])
        plsc.subcore_barrier()

        def chunk_body(ch, _):
            cbase = base_pt + ch * MC
            pltpu.sync_copy(rk_hbm.at[pl.ds(cbase, MC)], rk_mc)
            pltpu.sync_copy(wt_hbm.at[pl.ds(cbase, MC)], wt_mc)
            pltpu.sync_copy(px_hbm.at[pl.ds(cbase, MC)], px_mc)

            def build(sub, p):
                sb = sub * SUB
                for r in range(SUB // 16):
                    sl16 = pl.ds(sb + r * 16, 16)
                    rk128[p][pl.ds(r * 16, 16)] = rk_mc[sl16]
                    ix128[p][pl.ds(r * 16, 16)] = px_mc[sl16] + qoff
                return pltpu.async_copy(ftq_hbm.at[ix128[p]], rows[p],
                                        gsem[p])

            gd = [None, None]
            sd = [None, None]
            gd[0] = build(0, 0)
            for sub in range(NSUB):
                p = sub % 2
                o = 1 - p
                if sub + 1 < NSUB:
                    if sd[o] is not None:
                        sd[o].wait()
                        sd[o] = None
                    gd[o] = build(sub + 1, o)
                gd[p].wait()
                sb = sub * SUB

                def srow(i, _):
                    for u in range(4):
                        r = i * 4 + u
                        wr = plsc.load_gather(
                            wt_mc, [jnp.full((16,), sb + r, I32)])
                        rows[p][r, pl.ds(0, 16)] = (
                            rows[p][r, pl.ds(0, 16)] * wr)
                    return 0

                lax.fori_loop(0, SUB // 4, srow, 0)
                sd[p] = pltpu.async_copy(rows[p], acc.at[rk128[p]],
                                         ssem[p], add=True)
            for p in range(2):
                if sd[p] is not None:
                    sd[p].wait()
            return 0

        lax.fori_loop(0, SHARD_MC, chunk_body, 0)
        plsc.subcore_barrier()
        pltpu.sync_copy(acc.at[pl.ds(s * SLICE, SLICE)],
                        out_hbm.at[q, pl.ds(s * SLICE, SLICE)])


@functools.cache
def _sc_scatter():
    return pl.kernel(
        _sc_body,
        out_type=jax.ShapeDtypeStruct((NQ, NV, CQ), F32),
        mesh=plsc.VectorSubcoreMesh(core_axis_name="c", subcore_axis_name="s"),
        scratch_types=[
            pltpu.VMEM((MC,), I32),
            pltpu.VMEM((MC,), F32),
            pltpu.VMEM((MC,), I32),
            (pltpu.VMEM((SUB,), I32),) * 2,
            (pltpu.VMEM((SUB,), I32),) * 2,
            (pltpu.VMEM((SUB, CQ), F32),) * 2,
            pltpu.VMEM_SHARED((NV, CQ), F32),
            pltpu.SemaphoreType.DMA,
            pltpu.SemaphoreType.DMA,
            pltpu.SemaphoreType.DMA,
            pltpu.SemaphoreType.DMA,
        ],
        compiler_params=pltpu.CompilerParams(
            needs_layout_passes=False, use_tc_tiling_on_sc=False),
    )


def kernel(features, depth, lidar2img):
    gx, gy, gz = _geometry(lidar2img)
    depth_r = depth.reshape(BN, D, HW)
    wt, rk, px = _tc_phase(gx, gy, gz, depth_r)

    pad = P_PAD - P
    rk_f = jnp.concatenate([rk.reshape(P), jnp.zeros((pad,), I32)])
    wt_f = jnp.concatenate([wt.reshape(P), jnp.zeros((pad,), F32)])
    px_f = jnp.concatenate([px.reshape(P), jnp.zeros((pad,), I32)])
    ftq = (features.reshape(BN, C, HW).transpose(0, 2, 1).reshape(Q, NQ, CQ)
           .transpose(1, 0, 2).reshape(NQ * Q, CQ))
    zeros = jnp.zeros((SLICE, CQ), F32)

    out = _sc_scatter()(rk_f, wt_f, px_f, ftq, zeros)
    o = out.reshape(NQ, 200, 200, 2, CQ)
    return o.transpose(3, 0, 4, 1, 2).reshape(2, C, 200, 200)


# speedup vs baseline: 47.4857x; 3.2018x over previous
import functools

import jax
import jax.numpy as jnp
from jax import lax
from jax.experimental import pallas as pl
from jax.experimental.pallas import tpu as pltpu
from jax.experimental.pallas import tpu_sc as plsc

F32 = jnp.float32
I32 = jnp.int32

BN = 12
D = 41
FH, FW = 28, 60
HW = FH * FW
C = 64
P = BN * D * HW
Q = BN * HW

NQ = 4
CQ = C // NQ
NV = 80000
MC = 2048
SUB = 128
NSUB = MC // SUB
SHARD_MC = 26
SHARD = SHARD_MC * MC
P_PAD = 16 * SHARD
SLICE = NV // 16


def _tc_body(gx_ref, gy_ref, gz_ref, depth_ref, wt_ref, rk_ref, px_ref):
    i = pl.program_id(0)
    d = depth_ref[0]
    mx = jnp.max(d, axis=0, keepdims=True)
    e = jnp.exp(d - mx)
    wt = e / jnp.sum(e, axis=0, keepdims=True)

    vx = ((gx_ref[0] + 50.0) / 0.5).astype(I32)
    vy = ((gy_ref[0] + 50.0) / 0.5).astype(I32)
    vz = ((gz_ref[0] + 10.0) / 20.0).astype(I32)
    kept = ((vx >= 0) & (vx < 200) & (vy >= 0) & (vy < 200)
            & (vz >= 0) & (vz < 1))
    rank = vx * 400 + vy * 2 + i // 6
    rk_ref[0] = jnp.where(kept, rank, 0)
    wt_ref[0] = jnp.where(kept, wt, 0.0)
    px_ref[0] = lax.broadcasted_iota(I32, (D, HW), 1) + i * HW


def _tc_phase(gx, gy, gz, depth):
    bs = pl.BlockSpec((1, D, HW), lambda i: (i, 0, 0))
    return pl.pallas_call(
        _tc_body,
        grid=(BN,),
        in_specs=[bs, bs, bs, bs],
        out_specs=[bs, bs, bs],
        out_shape=[
            jax.ShapeDtypeStruct((BN, D, HW), F32),
            jax.ShapeDtypeStruct((BN, D, HW), I32),
            jax.ShapeDtypeStruct((BN, D, HW), I32),
        ],
    )(gx, gy, gz, depth)


def _geometry(lidar2img):
    ds = jnp.arange(4.0, 45.0, 1.0, dtype=F32).reshape(-1, 1, 1)
    ds = jnp.broadcast_to(ds, (D, FH, FW))
    xs = jnp.broadcast_to(
        jnp.linspace(0.0, 479.0, FW, dtype=F32).reshape(1, 1, FW), (D, FH, FW))
    ys = jnp.broadcast_to(
        jnp.linspace(0.0, 223.0, FH, dtype=F32).reshape(1, FH, 1), (D, FH, FW))
    frustum = jnp.stack((xs, ys, ds), -1)
    eps = 1e-05
    pts = jnp.broadcast_to(frustum[None, None], (2, 6, D, FH, FW, 3))
    pts = jnp.concatenate([pts, jnp.ones_like(pts[..., :1])], -1)
    xy = pts[..., :2] * jnp.maximum(pts[..., 2:3], jnp.ones_like(pts[..., 2:3]) * eps)
    pts = jnp.concatenate([xy, pts[..., 2:]], -1)
    img2lidars = jnp.linalg.inv(lidar2img)
    geom = jnp.einsum('bnij,bndhwj->bndhwi', img2lidars, pts)[..., :3]
    g = geom.reshape(BN, D, HW, 3)
    return g[..., 0], g[..., 1], g[..., 2]


def _sc_body(rk_hbm, wt_hbm, px_hbm, ftq_hbm, z_hbm, cf_hbm, sf_hbm,
             out_hbm, rk_mc, wt_mc, px_mc, ix128, rows, cf_v, sf_v, acc,
             gsem):
    core = lax.axis_index("c")
    s = lax.axis_index("s")
    base_row = s * (SHARD // SUB)

    pltpu.sync_copy(cf_hbm.at[s], cf_v)
    pltpu.sync_copy(sf_hbm.at[s], sf_v)

    for qq in range(NQ // 2):
        q = core * 2 + qq
        qoff = q * Q
        pltpu.sync_copy(z_hbm, acc.at[pl.ds(s * SLICE, SLICE)])
        plsc.subcore_barrier()

        def chunk_body(ch, _):
            cfl = plsc.load_gather(cf_v, [jnp.full((16,), ch, I32)])[0]

            @pl.when(cfl > 0)
            def _():
                crow = base_row + ch * NSUB
                pltpu.sync_copy(rk_hbm.at[pl.ds(crow, NSUB)], rk_mc)
                pltpu.sync_copy(wt_hbm.at[pl.ds(crow, NSUB)], wt_mc)
                pltpu.sync_copy(px_hbm.at[pl.ds(crow, NSUB)], px_mc)
                for sub in range(NSUB):
                    sfl = plsc.load_gather(
                        sf_v, [ch * 16 + jnp.full((16,), sub, I32)])[0]

                    @pl.when(sfl > 0)
                    def _():
                        for r in range(SUB // 16):
                            sl16 = pl.ds(r * 16, 16)
                            ix128[sl16] = px_mc[sub, sl16] + qoff
                        pltpu.async_copy(ftq_hbm.at[ix128], rows, gsem).wait()

                        def srow(i, _):
                            for u in range(4):
                                r = i * 4 + u
                                wr = plsc.load_gather(
                                    wt_mc, [jnp.full((16,), sub, I32),
                                            jnp.full((16,), r, I32)])
                                rows[r, pl.ds(0, 16)] = (
                                    rows[r, pl.ds(0, 16)] * wr)
                            return 0

                        lax.fori_loop(0, SUB // 4, srow, 0)
                        pltpu.sync_copy(rows, acc.at[rk_mc.at[sub]], add=True)
            return 0

        lax.fori_loop(0, SHARD_MC, chunk_body, 0)
        plsc.subcore_barrier()
        pltpu.sync_copy(acc.at[pl.ds(s * SLICE, SLICE)],
                        out_hbm.at[q, pl.ds(s * SLICE, SLICE)])


@functools.cache
def _sc_scatter():
    return pl.kernel(
        _sc_body,
        out_type=jax.ShapeDtypeStruct((NQ, NV, CQ), F32),
        mesh=plsc.VectorSubcoreMesh(core_axis_name="c", subcore_axis_name="s"),
        scratch_types=[
            pltpu.VMEM((NSUB, SUB), I32),
            pltpu.VMEM((NSUB, SUB), F32),
            pltpu.VMEM((NSUB, SUB), I32),
            pltpu.VMEM((SUB,), I32),
            pltpu.VMEM((SUB, CQ), F32),
            pltpu.VMEM((32,), I32),
            pltpu.VMEM((SHARD_MC * NSUB + 96,), I32),
            pltpu.VMEM_SHARED((NV, CQ), F32),
            pltpu.SemaphoreType.DMA,
        ],
        compiler_params=pltpu.CompilerParams(
            needs_layout_passes=False, use_tc_tiling_on_sc=False),
    )


def kernel(features, depth, lidar2img):
    gx, gy, gz = _geometry(lidar2img)
    depth_r = depth.reshape(BN, D, HW)
    wt, rk, px = _tc_phase(gx, gy, gz, depth_r)

    pad = P_PAD - P
    rk_f = jnp.concatenate([rk.reshape(P), jnp.zeros((pad,), I32)])
    wt_f = jnp.concatenate([wt.reshape(P), jnp.zeros((pad,), F32)])
    px_f = jnp.concatenate([px.reshape(P), jnp.zeros((pad,), I32)])
    rk2 = rk_f.reshape(P_PAD // SUB, SUB)
    wt2 = wt_f.reshape(P_PAD // SUB, SUB)
    px2 = px_f.reshape(P_PAD // SUB, SUB)
    vsub = (wt_f > 0).reshape(16, SHARD_MC, NSUB, SUB).any(-1)
    cf = jnp.pad(vsub.any(-1).astype(I32), ((0, 0), (0, 32 - SHARD_MC)))
    sf = jnp.pad(vsub.reshape(16, SHARD_MC * NSUB).astype(I32),
                 ((0, 0), (0, 96)))
    ftq = (features.reshape(BN, C, HW).transpose(0, 2, 1).reshape(Q, NQ, CQ)
           .transpose(1, 0, 2).reshape(NQ * Q, CQ))
    zeros = jnp.zeros((SLICE, CQ), F32)

    out = _sc_scatter()(rk2, wt2, px2, ftq, zeros, cf, sf)
    o = out.reshape(NQ, 200, 200, 2, CQ)
    return o.transpose(3, 0, 4, 1, 2).reshape(2, C, 200, 200)


# speedup vs baseline: 75.7103x; 1.5944x over previous
import functools

import jax
import jax.numpy as jnp
from jax import lax
from jax.experimental import pallas as pl
from jax.experimental.pallas import tpu as pltpu
from jax.experimental.pallas import tpu_sc as plsc

F32 = jnp.float32
I32 = jnp.int32

BN = 12
D = 41
FH, FW = 28, 60
HW = FH * FW
C = 64
P = BN * D * HW
Q = BN * HW

NQ = 4
CQ = C // NQ
NV = 80000
NVP = 81920
MC = 2048
SUB = 128
NSUB = MC // SUB
SHARD_MC = 26
SHARD = SHARD_MC * MC
P_PAD = 16 * SHARD
SLICE = NVP // 16
BLK = 1024


def _tc_body(gx_ref, gy_ref, gz_ref, depth_ref, wt_ref, rk_ref, px_ref):
    i = pl.program_id(0)
    d = depth_ref[0]
    mx = jnp.max(d, axis=0, keepdims=True)
    e = jnp.exp(d - mx)
    wt = e / jnp.sum(e, axis=0, keepdims=True)

    vx = ((gx_ref[0] + 50.0) / 0.5).astype(I32)
    vy = ((gy_ref[0] + 50.0) / 0.5).astype(I32)
    vz = ((gz_ref[0] + 10.0) / 20.0).astype(I32)
    kept = ((vx >= 0) & (vx < 200) & (vy >= 0) & (vy < 200)
            & (vz >= 0) & (vz < 1))
    rank = vx * 200 + vy + (i // 6) * 40960
    rk_ref[0] = jnp.where(kept, rank, 0)
    wt_ref[0] = jnp.where(kept, wt, 0.0)
    px_ref[0] = lax.broadcasted_iota(I32, (D, HW), 1) + i * HW


def _tc_phase(gx, gy, gz, depth):
    bs = pl.BlockSpec((1, D, HW), lambda i: (i, 0, 0))
    return pl.pallas_call(
        _tc_body,
        grid=(BN,),
        in_specs=[bs, bs, bs, bs],
        out_specs=[bs, bs, bs],
        out_shape=[
            jax.ShapeDtypeStruct((BN, D, HW), F32),
            jax.ShapeDtypeStruct((BN, D, HW), I32),
            jax.ShapeDtypeStruct((BN, D, HW), I32),
        ],
    )(gx, gy, gz, depth)


def _geometry(lidar2img):
    ds = jnp.arange(4.0, 45.0, 1.0, dtype=F32).reshape(-1, 1, 1)
    ds = jnp.broadcast_to(ds, (D, FH, FW))
    xs = jnp.broadcast_to(
        jnp.linspace(0.0, 479.0, FW, dtype=F32).reshape(1, 1, FW), (D, FH, FW))
    ys = jnp.broadcast_to(
        jnp.linspace(0.0, 223.0, FH, dtype=F32).reshape(1, FH, 1), (D, FH, FW))
    frustum = jnp.stack((xs, ys, ds), -1)
    eps = 1e-05
    pts = jnp.broadcast_to(frustum[None, None], (2, 6, D, FH, FW, 3))
    pts = jnp.concatenate([pts, jnp.ones_like(pts[..., :1])], -1)
    xy = pts[..., :2] * jnp.maximum(pts[..., 2:3], jnp.ones_like(pts[..., 2:3]) * eps)
    pts = jnp.concatenate([xy, pts[..., 2:]], -1)
    img2lidars = jnp.linalg.inv(lidar2img)
    geom = jnp.einsum('bnij,bndhwj->bndhwi', img2lidars, pts)[..., :3]
    g = geom.reshape(BN, D, HW, 3)
    return g[..., 0], g[..., 1], g[..., 2]


def _sc_body(rk_hbm, wt_hbm, px_hbm, ftq_hbm, z_hbm, cf_hbm, sf_hbm,
             out_hbm, rk_mc, wt_mc, px_mc, ix128, rows, cf_v, sf_v, acc,
             tbuf, tbuf_t, gsem):
    core = lax.axis_index("c")
    s = lax.axis_index("s")
    base_row = s * (SHARD // SUB)
    bb = s // 8
    boff = (s % 8) * SLICE

    pltpu.sync_copy(cf_hbm.at[s], cf_v)
    pltpu.sync_copy(sf_hbm.at[s], sf_v)

    for qq in range(NQ // 2):
        q = core * 2 + qq
        qoff = q * Q
        pltpu.sync_copy(z_hbm, acc.at[pl.ds(s * SLICE, SLICE)])
        plsc.subcore_barrier()

        def chunk_body(ch, _):
            cfl = plsc.load_gather(cf_v, [jnp.full((16,), ch, I32)])[0]

            @pl.when(cfl > 0)
            def _():
                crow = base_row + ch * NSUB
                pltpu.sync_copy(rk_hbm.at[pl.ds(crow, NSUB)], rk_mc)
                pltpu.sync_copy(wt_hbm.at[pl.ds(crow, NSUB)], wt_mc)
                pltpu.sync_copy(px_hbm.at[pl.ds(crow, NSUB)], px_mc)
                for sub in range(NSUB):
                    sfl = plsc.load_gather(
                        sf_v, [ch * 16 + jnp.full((16,), sub, I32)])[0]

                    @pl.when(sfl > 0)
                    def _():
                        for r in range(SUB // 16):
                            sl16 = pl.ds(r * 16, 16)
                            ix128[sl16] = px_mc[sub, sl16] + qoff
                        pltpu.async_copy(ftq_hbm.at[ix128], rows, gsem).wait()

                        def srow(i, _):
                            for u in range(4):
                                r = i * 4 + u
                                wr = plsc.load_gather(
                                    wt_mc, [jnp.full((16,), sub, I32),
                                            jnp.full((16,), r, I32)])
                                rows[r, pl.ds(0, 16)] = (
                                    rows[r, pl.ds(0, 16)] * wr)
                            return 0

                        lax.fori_loop(0, SUB // 4, srow, 0)
                        pltpu.sync_copy(rows, acc.at[rk_mc.at[sub]], add=True)
            return 0

        lax.fori_loop(0, SHARD_MC, chunk_body, 0)
        plsc.subcore_barrier()
        for h in range(SLICE // BLK):
            pltpu.sync_copy(
                acc.at[pl.ds(s * SLICE + h * BLK, BLK)], tbuf)

            def trow(i0, _):
                ii = i0 * 16 + lax.iota(I32, 16)
                for cc in range(CQ):
                    v = plsc.load_gather(
                        tbuf, [ii, jnp.full((16,), cc, I32)])
                    tbuf_t[cc, pl.ds(i0 * 16, 16)] = v
                return 0

            lax.fori_loop(0, BLK // 16, trow, 0)
            for cc in range(CQ):
                pltpu.sync_copy(
                    tbuf_t.at[cc],
                    out_hbm.at[bb, q, cc,
                               pl.ds(boff + h * BLK, BLK)])


@functools.cache
def _sc_scatter():
    return pl.kernel(
        _sc_body,
        out_type=jax.ShapeDtypeStruct((2, NQ, CQ, 8 * SLICE), F32),
        mesh=plsc.VectorSubcoreMesh(core_axis_name="c", subcore_axis_name="s"),
        scratch_types=[
            pltpu.VMEM((NSUB, SUB), I32),
            pltpu.VMEM((NSUB, SUB), F32),
            pltpu.VMEM((NSUB, SUB), I32),
            pltpu.VMEM((SUB,), I32),
            pltpu.VMEM((SUB, CQ), F32),
            pltpu.VMEM((32,), I32),
            pltpu.VMEM((SHARD_MC * NSUB + 96,), I32),
            pltpu.VMEM_SHARED((NVP, CQ), F32),
            pltpu.VMEM((BLK, CQ), F32),
            pltpu.VMEM((CQ, BLK), F32),
            pltpu.SemaphoreType.DMA,
        ],
        compiler_params=pltpu.CompilerParams(
            needs_layout_passes=False, use_tc_tiling_on_sc=False),
    )


def kernel(features, depth, lidar2img):
    gx, gy, gz = _geometry(lidar2img)
    pass
    depth_r = depth.reshape(BN, D, HW)
    wt, rk, px = _tc_phase(gx, gy, gz, depth_r)

    pad = P_PAD - P
    rk_f = jnp.concatenate([rk.reshape(P), jnp.zeros((pad,), I32)])
    wt_f = jnp.concatenate([wt.reshape(P), jnp.zeros((pad,), F32)])
    px_f = jnp.concatenate([px.reshape(P), jnp.zeros((pad,), I32)])
    rk2 = rk_f.reshape(P_PAD // SUB, SUB)
    wt2 = wt_f.reshape(P_PAD // SUB, SUB)
    px2 = px_f.reshape(P_PAD // SUB, SUB)
    vsub = (wt_f > 0).reshape(16, SHARD_MC, NSUB, SUB).any(-1)
    cf = jnp.pad(vsub.any(-1).astype(I32), ((0, 0), (0, 32 - SHARD_MC)))
    sf = jnp.pad(vsub.reshape(16, SHARD_MC * NSUB).astype(I32),
                 ((0, 0), (0, 96)))
    ftq = (features.reshape(BN, C, HW).transpose(0, 2, 1).reshape(Q, NQ, CQ)
           .transpose(1, 0, 2).reshape(NQ * Q, CQ))
    zeros = jnp.zeros((SLICE, CQ), F32)

    out = _sc_scatter()(rk2, wt2, px2, ftq, zeros, cf, sf)
    return out[..., :40000].reshape(2, C, 200, 200)


# speedup vs baseline: 79.1683x; 1.0457x over previous
import functools

import jax
import jax.numpy as jnp
from jax import lax
from jax.experimental import pallas as pl
from jax.experimental.pallas import tpu as pltpu
from jax.experimental.pallas import tpu_sc as plsc

F32 = jnp.float32
I32 = jnp.int32

BN = 12
D = 41
FH, FW = 28, 60
HW = FH * FW
C = 64
P = BN * D * HW
Q = BN * HW

NQ = 4
CQ = C // NQ
NV = 80000
NVP = 81920
MC = 2048
SUB = 128
NSUB = MC // SUB
SHARD_MC = 26
SHARD = SHARD_MC * MC
P_PAD = 16 * SHARD
SLICE = NVP // 16
BLK = 1024


def _tc_body(gx_ref, gy_ref, gz_ref, depth_ref, wt_ref, pk_ref):
    i = pl.program_id(0)
    d = depth_ref[0]
    mx = jnp.max(d, axis=0, keepdims=True)
    e = jnp.exp(d - mx)
    wt = e / jnp.sum(e, axis=0, keepdims=True)

    vx = ((gx_ref[0] + 50.0) / 0.5).astype(I32)
    vy = ((gy_ref[0] + 50.0) / 0.5).astype(I32)
    vz = ((gz_ref[0] + 10.0) / 20.0).astype(I32)
    kept = ((vx >= 0) & (vx < 200) & (vy >= 0) & (vy < 200)
            & (vz >= 0) & (vz < 1))
    rank = vx * 200 + vy + (i // 6) * 40960
    px = lax.broadcasted_iota(I32, (D, HW), 1) + i * HW
    pk_ref[0] = px | (jnp.where(kept, rank, 0) << 15)
    wt_ref[0] = jnp.where(kept, wt, 0.0)


def _tc_phase(gx, gy, gz, depth):
    bs = pl.BlockSpec((1, D, HW), lambda i: (i, 0, 0))
    return pl.pallas_call(
        _tc_body,
        grid=(BN,),
        in_specs=[bs, bs, bs, bs],
        out_specs=[bs, bs],
        out_shape=[
            jax.ShapeDtypeStruct((BN, D, HW), F32),
            jax.ShapeDtypeStruct((BN, D, HW), I32),
        ],
    )(gx, gy, gz, depth)


def _geometry(lidar2img):
    ds = jnp.arange(4.0, 45.0, 1.0, dtype=F32).reshape(-1, 1, 1)
    ds = jnp.broadcast_to(ds, (D, FH, FW))
    xs = jnp.broadcast_to(
        jnp.linspace(0.0, 479.0, FW, dtype=F32).reshape(1, 1, FW), (D, FH, FW))
    ys = jnp.broadcast_to(
        jnp.linspace(0.0, 223.0, FH, dtype=F32).reshape(1, FH, 1), (D, FH, FW))
    frustum = jnp.stack((xs, ys, ds), -1)
    eps = 1e-05
    pts = jnp.broadcast_to(frustum[None, None], (2, 6, D, FH, FW, 3))
    pts = jnp.concatenate([pts, jnp.ones_like(pts[..., :1])], -1)
    xy = pts[..., :2] * jnp.maximum(pts[..., 2:3], jnp.ones_like(pts[..., 2:3]) * eps)
    pts = jnp.concatenate([xy, pts[..., 2:]], -1)
    img2lidars = jnp.linalg.inv(lidar2img)
    geom = jnp.einsum('bnij,bndhwj->bndhwi', img2lidars, pts)[..., :3]
    g = geom.reshape(BN, D, HW, 3)
    return g[..., 0], g[..., 1], g[..., 2]


def _sc_body(pk_hbm, wt_hbm, ftq_hbm, cf_hbm, sf_hbm,
             out_hbm, pk_mc, wt_mc, rk128, ix128, rows, cf_v, sf_v, acc,
             tbuf, tbuf_t, gsem):
    core = lax.axis_index("c")
    s = lax.axis_index("s")
    base_row = s * (SHARD // SUB)
    bb = s // 8
    boff = (s % 8) * SLICE

    pltpu.sync_copy(cf_hbm.at[s], cf_v)
    pltpu.sync_copy(sf_hbm.at[s], sf_v)

    def zrow(i0, _):
        tbuf[i0, pl.ds(0, 16)] = jnp.zeros((16,), F32)
        return 0

    lax.fori_loop(0, BLK, zrow, 0)

    for qq in range(NQ // 2):
        q = core * 2 + qq
        qoff = q * Q
        for h in range(SLICE // BLK):
            pltpu.sync_copy(tbuf, acc.at[pl.ds(s * SLICE + h * BLK, BLK)])
        plsc.subcore_barrier()

        def chunk_body(ch, _):
            cfl = plsc.load_gather(cf_v, [jnp.full((16,), ch, I32)])[0]

            @pl.when(cfl > 0)
            def _():
                crow = base_row + ch * NSUB
                pltpu.sync_copy(pk_hbm.at[pl.ds(crow, NSUB)], pk_mc)
                pltpu.sync_copy(wt_hbm.at[pl.ds(crow, NSUB)], wt_mc)
                for sub in range(NSUB):
                    sfl = plsc.load_gather(
                        sf_v, [ch * 16 + jnp.full((16,), sub, I32)])[0]

                    @pl.when(sfl > 0)
                    def _():
                        for r in range(SUB // 16):
                            sl16 = pl.ds(r * 16, 16)
                            v = pk_mc[sub, sl16]
                            ix128[sl16] = (v & 0x7FFF) + qoff
                            rk128[sl16] = lax.shift_right_logical(v, 15)
                        pltpu.async_copy(ftq_hbm.at[ix128], rows, gsem).wait()

                        def srow(i, _):
                            for u in range(4):
                                r = i * 4 + u
                                wr = plsc.load_gather(
                                    wt_mc, [jnp.full((16,), sub, I32),
                                            jnp.full((16,), r, I32)])
                                rows[r, pl.ds(0, 16)] = (
                                    rows[r, pl.ds(0, 16)] * wr)
                            return 0

                        lax.fori_loop(0, SUB // 4, srow, 0)
                        pltpu.sync_copy(rows, acc.at[rk128], add=True)
            return 0

        lax.fori_loop(0, SHARD_MC, chunk_body, 0)
        plsc.subcore_barrier()
        for h in range(SLICE // BLK):
            pltpu.sync_copy(
                acc.at[pl.ds(s * SLICE + h * BLK, BLK)], tbuf)

            def trow(i0, _):
                ii = i0 * 16 + lax.iota(I32, 16)
                for cc in range(CQ):
                    v = plsc.load_gather(
                        tbuf, [ii, jnp.full((16,), cc, I32)])
                    tbuf_t[cc, pl.ds(i0 * 16, 16)] = v
                return 0

            lax.fori_loop(0, BLK // 16, trow, 0)
            for cc in range(CQ):
                pltpu.sync_copy(
                    tbuf_t.at[cc],
                    out_hbm.at[bb, q, cc,
                               pl.ds(boff + h * BLK, BLK)])


@functools.cache
def _sc_scatter():
    return pl.kernel(
        _sc_body,
        out_type=jax.ShapeDtypeStruct((2, NQ, CQ, 8 * SLICE), F32),
        mesh=plsc.VectorSubcoreMesh(core_axis_name="c", subcore_axis_name="s"),
        scratch_types=[
            pltpu.VMEM((NSUB, SUB), I32),
            pltpu.VMEM((NSUB, SUB), F32),
            pltpu.VMEM((SUB,), I32),
            pltpu.VMEM((SUB,), I32),
            pltpu.VMEM((SUB, CQ), F32),
            pltpu.VMEM((32,), I32),
            pltpu.VMEM((SHARD_MC * NSUB + 96,), I32),
            pltpu.VMEM_SHARED((NVP, CQ), F32),
            pltpu.VMEM((BLK, CQ), F32),
            pltpu.VMEM((CQ, BLK), F32),
            pltpu.SemaphoreType.DMA,
        ],
        compiler_params=pltpu.CompilerParams(
            needs_layout_passes=False, use_tc_tiling_on_sc=False),
    )


def kernel(features, depth, lidar2img):
    gx, gy, gz = _geometry(lidar2img)
    pass
    depth_r = depth.reshape(BN, D, HW)
    wt, pk = _tc_phase(gx, gy, gz, depth_r)

    pad = P_PAD - P
    pk_f = jnp.concatenate([pk.reshape(P), jnp.zeros((pad,), I32)])
    wt_f = jnp.concatenate([wt.reshape(P), jnp.zeros((pad,), F32)])
    pk2 = pk_f.reshape(P_PAD // SUB, SUB)
    wt2 = wt_f.reshape(P_PAD // SUB, SUB)
    vsub = (wt_f > 0).reshape(16, SHARD_MC, NSUB, SUB).any(-1)
    cf = jnp.pad(vsub.any(-1).astype(I32), ((0, 0), (0, 32 - SHARD_MC)))
    sf = jnp.pad(vsub.reshape(16, SHARD_MC * NSUB).astype(I32),
                 ((0, 0), (0, 96)))
    ftq = (features.reshape(BN, C, HW).transpose(0, 2, 1).reshape(Q, NQ, CQ)
           .transpose(1, 0, 2).reshape(NQ * Q, CQ))
    out = _sc_scatter()(pk2, wt2, ftq, cf, sf)
    return out[..., :40000].reshape(2, C, 200, 200)
